# Initial kernel scaffold; baseline (speedup 1.0000x reference)
import jax
import jax.numpy as jnp
from jax.experimental import pallas as pl


def kernel(node_features, frac_coords, lattices, edge_index, edge2graph, W1, b1, W2, b2, W3, b3, W4, b4):
    raise NotImplementedError("write your pallas kernel here")


# speedup vs baseline: 2.2106x; 2.2106x over previous
import functools

import jax
import jax.numpy as jnp
from jax import lax
from jax.experimental import pallas as pl
from jax.experimental.pallas import tpu as pltpu
from jax.experimental.pallas import tpu_sc as plsc

_N = 10000
_E = 320000
_G = 256
_H = 128

_NC = 2
_NS = 16
_NW = _NC * _NS
_EPW = _E // _NW
_W = 80

_BE = 2000
_BN = 1000


@functools.lru_cache(maxsize=None)
def _mesh():
    return plsc.VectorSubcoreMesh(core_axis_name="c", subcore_axis_name="s")


def _pre_body(nf_ref, lat9_ref, w1a_ref, w1b_ref, w1c_ref, b1_ref,
              p_ref, q_ref, rb_ref):
    x = nf_ref[...]
    p_ref[...] = jnp.dot(x, w1a_ref[...], preferred_element_type=jnp.float32)
    q_ref[...] = jnp.dot(x, w1b_ref[...], preferred_element_type=jnp.float32)

    @pl.when(pl.program_id(0) == 0)
    def _():
        lat9 = lat9_ref[...]
        cols = [lat9[:, k:k + 1] for k in range(9)]
        acc = jnp.broadcast_to(b1_ref[...], (_G, _H))
        for i in range(3):
            for j in range(3):
                ip = (cols[3 * i + 0] * cols[3 * j + 0]
                      + cols[3 * i + 1] * cols[3 * j + 1]
                      + cols[3 * i + 2] * cols[3 * j + 2])
                acc = acc + ip * w1c_ref[3 * i + j:3 * i + j + 1, :]
        rb_ref[...] = acc


def _pre(nf, lat9, w1a, w1b, w1c, b1r):
    grid = (_N // _BN,)
    return pl.pallas_call(
        _pre_body,
        grid=grid,
        in_specs=[
            pl.BlockSpec((_BN, _H), lambda i: (i, 0)),
            pl.BlockSpec((_G, 9), lambda i: (0, 0)),
            pl.BlockSpec((_H, _H), lambda i: (0, 0)),
            pl.BlockSpec((_H, _H), lambda i: (0, 0)),
            pl.BlockSpec((9, _H), lambda i: (0, 0)),
            pl.BlockSpec((1, _H), lambda i: (0, 0)),
        ],
        out_specs=[
            pl.BlockSpec((_BN, _H), lambda i: (i, 0)),
            pl.BlockSpec((_BN, _H), lambda i: (i, 0)),
            pl.BlockSpec((_G, _H), lambda i: (0, 0)),
        ],
        out_shape=[
            jax.ShapeDtypeStruct((_N, _H), jnp.float32),
            jax.ShapeDtypeStruct((_N, _H), jnp.float32),
            jax.ShapeDtypeStruct((_G, _H), jnp.float32),
        ],
    )(nf, lat9, w1a, w1b, w1c, b1r)


def _gather_body(p_hbm, q_hbm, rb_hbm, src_hbm, dst_hbm, e2g_hbm,
                 a_hbm, b_hbm, c_hbm,
                 si_v, di_v, gi_v, a_v, b_v, c_v, sem):
    wid = lax.axis_index("s") * _NC + lax.axis_index("c")
    base = wid * _EPW

    @pl.loop(0, _EPW, step=_W)
    def _(off):
        st = base + off
        pltpu.sync_copy(src_hbm.at[pl.ds(st, _W)], si_v)
        pltpu.sync_copy(dst_hbm.at[pl.ds(st, _W)], di_v)
        pltpu.sync_copy(e2g_hbm.at[pl.ds(st, _W)], gi_v)
        c1 = pltpu.async_copy(p_hbm.at[si_v], a_v, sem)
        c2 = pltpu.async_copy(q_hbm.at[di_v], b_v, sem)
        c3 = pltpu.async_copy(rb_hbm.at[gi_v], c_v, sem)
        c1.wait()
        c2.wait()
        c3.wait()
        pltpu.sync_copy(a_v, a_hbm.at[pl.ds(st, _W)])
        pltpu.sync_copy(b_v, b_hbm.at[pl.ds(st, _W)])
        pltpu.sync_copy(c_v, c_hbm.at[pl.ds(st, _W)])


@functools.lru_cache(maxsize=None)
def _gather_kernel():
    return pl.kernel(
        _gather_body,
        mesh=_mesh(),
        out_type=(
            jax.ShapeDtypeStruct((_E, _H), jnp.float32),
            jax.ShapeDtypeStruct((_E, _H), jnp.float32),
            jax.ShapeDtypeStruct((_E, _H), jnp.float32),
        ),
        scratch_types=[
            pltpu.VMEM((_W,), jnp.int32),
            pltpu.VMEM((_W,), jnp.int32),
            pltpu.VMEM((_W,), jnp.int32),
            pltpu.VMEM((_W, _H), jnp.float32),
            pltpu.VMEM((_W, _H), jnp.float32),
            pltpu.VMEM((_W, _H), jnp.float32),
            pltpu.SemaphoreType.DMA,
        ],
    )


def _aux_body(fc_hbm, src_hbm, dst_hbm, ones_hbm, zc_hbm,
              xi_hbm, xj_hbm, cnt_hbm,
              si_v, di_v, xi_v, xj_v, ones_v, cacc_sh, sem):
    cid = lax.axis_index("c")
    sid = lax.axis_index("s")
    wid = sid * _NC + cid
    base = wid * _EPW

    @pl.when(sid == 0)
    def _():
        pltpu.sync_copy(zc_hbm, cacc_sh)
    pltpu.sync_copy(ones_hbm, ones_v)
    plsc.subcore_barrier()

    @pl.loop(0, _EPW, step=_W)
    def _(off):
        st = base + off
        pltpu.sync_copy(src_hbm.at[pl.ds(st, _W)], si_v)
        pltpu.sync_copy(dst_hbm.at[pl.ds(st, _W)], di_v)
        c1 = pltpu.async_copy(fc_hbm.at[si_v], xi_v, sem)
        c2 = pltpu.async_copy(fc_hbm.at[di_v], xj_v, sem)
        c1.wait()
        c2.wait()
        pltpu.sync_copy(xi_v, xi_hbm.at[pl.ds(st, _W)])
        pltpu.sync_copy(xj_v, xj_hbm.at[pl.ds(st, _W)])
        pltpu.sync_copy(ones_v, cacc_sh.at[si_v], add=True)

    plsc.subcore_barrier()

    @pl.when(sid == 0)
    def _():
        pltpu.sync_copy(cacc_sh, cnt_hbm.at[cid])


@functools.lru_cache(maxsize=None)
def _aux_kernel():
    return pl.kernel(
        _aux_body,
        mesh=_mesh(),
        out_type=(
            jax.ShapeDtypeStruct((_E, 16), jnp.float32),
            jax.ShapeDtypeStruct((_E, 16), jnp.float32),
            jax.ShapeDtypeStruct((_NC, _N, 16), jnp.float32),
        ),
        scratch_types=[
            pltpu.VMEM((_W,), jnp.int32),
            pltpu.VMEM((_W,), jnp.int32),
            pltpu.VMEM((_W, 16), jnp.float32),
            pltpu.VMEM((_W, 16), jnp.float32),
            pltpu.VMEM((_W, 16), jnp.float32),
            pltpu.VMEM_SHARED((_N, 16), jnp.float32),
            pltpu.SemaphoreType.DMA,
        ],
        compiler_params=pltpu.CompilerParams(use_tc_tiling_on_sc=False),
    )


def _edge_body(a_ref, b_ref, c_ref, xi_ref, xj_ref, w1dp_ref, w2_ref, b2_ref,
               out_ref):
    z = a_ref[...] + b_ref[...] + c_ref[...]
    d = xj_ref[...] - xi_ref[...]
    fd = jnp.where(d < 0.0, d + 1.0, d)
    z = z + jnp.dot(fd, w1dp_ref[...], preferred_element_type=jnp.float32)
    u = z * jax.nn.sigmoid(z)
    h2 = jnp.dot(u, w2_ref[...], preferred_element_type=jnp.float32) + b2_ref[...]
    out_ref[...] = h2 * jax.nn.sigmoid(h2)


def _edge(a, b, c, xi, xj, w1dp, w2, b2r):
    grid = (_E // _BE,)
    return pl.pallas_call(
        _edge_body,
        grid=grid,
        in_specs=[
            pl.BlockSpec((_BE, _H), lambda i: (i, 0)),
            pl.BlockSpec((_BE, _H), lambda i: (i, 0)),
            pl.BlockSpec((_BE, _H), lambda i: (i, 0)),
            pl.BlockSpec((_BE, 16), lambda i: (i, 0)),
            pl.BlockSpec((_BE, 16), lambda i: (i, 0)),
            pl.BlockSpec((16, _H), lambda i: (0, 0)),
            pl.BlockSpec((_H, _H), lambda i: (0, 0)),
            pl.BlockSpec((1, _H), lambda i: (0, 0)),
        ],
        out_specs=pl.BlockSpec((_BE, _H), lambda i: (i, 0)),
        out_shape=jax.ShapeDtypeStruct((_E, _H), jnp.float32),
    )(a, b, c, xi, xj, w1dp, w2, b2r)


def _scatter_body(ef_hbm, src_hbm, z_hbm, part_hbm, idx_v, rows_v, acc_sh,
                  sem):
    cid = lax.axis_index("c")
    sid = lax.axis_index("s")
    wid = sid * _NC + cid
    base = wid * _EPW

    @pl.when(sid == 0)
    def _():
        pltpu.sync_copy(z_hbm, acc_sh)
    plsc.subcore_barrier()

    @pl.loop(0, _EPW, step=_W)
    def _(off):
        st = base + off
        pltpu.sync_copy(src_hbm.at[pl.ds(st, _W)], idx_v)
        pltpu.sync_copy(ef_hbm.at[pl.ds(st, _W)], rows_v)
        pltpu.sync_copy(rows_v, acc_sh.at[idx_v], add=True)

    plsc.subcore_barrier()

    @pl.when(sid == 0)
    def _():
        pltpu.sync_copy(acc_sh, part_hbm.at[cid])


@functools.lru_cache(maxsize=None)
def _scatter_kernel():
    return pl.kernel(
        _scatter_body,
        mesh=_mesh(),
        out_type=jax.ShapeDtypeStruct((_NC, _N, _H), jnp.float32),
        scratch_types=[
            pltpu.VMEM((_W,), jnp.int32),
            pltpu.VMEM((_W, _H), jnp.float32),
            pltpu.VMEM_SHARED((_N, _H), jnp.float32),
            pltpu.SemaphoreType.DMA,
        ],
    )


def _node_body(part_ref, cnt_ref, nf_ref, w3a_ref, w3b_ref, b3_ref, w4_ref,
               b4_ref, out_ref):
    s = part_ref[0] + part_ref[1]
    cnt = cnt_ref[0][:, 0:1] + cnt_ref[1][:, 0:1]
    agg = s / jnp.maximum(cnt, 1.0)
    x = nf_ref[...]
    h = (jnp.dot(x, w3a_ref[...], preferred_element_type=jnp.float32)
         + jnp.dot(agg, w3b_ref[...], preferred_element_type=jnp.float32)
         + b3_ref[...])
    u = h * jax.nn.sigmoid(h)
    h2 = jnp.dot(u, w4_ref[...], preferred_element_type=jnp.float32) + b4_ref[...]
    out_ref[...] = x + h2 * jax.nn.sigmoid(h2)


def _node(part, cnt, nf, w3a, w3b, b3r, w4, b4r):
    grid = (_N // _BN,)
    return pl.pallas_call(
        _node_body,
        grid=grid,
        in_specs=[
            pl.BlockSpec((_NC, _BN, _H), lambda i: (0, i, 0)),
            pl.BlockSpec((_NC, _BN, 16), lambda i: (0, i, 0)),
            pl.BlockSpec((_BN, _H), lambda i: (i, 0)),
            pl.BlockSpec((_H, _H), lambda i: (0, 0)),
            pl.BlockSpec((_H, _H), lambda i: (0, 0)),
            pl.BlockSpec((1, _H), lambda i: (0, 0)),
            pl.BlockSpec((_H, _H), lambda i: (0, 0)),
            pl.BlockSpec((1, _H), lambda i: (0, 0)),
        ],
        out_specs=pl.BlockSpec((_BN, _H), lambda i: (i, 0)),
        out_shape=jax.ShapeDtypeStruct((_N, _H), jnp.float32),
    )(part, cnt, nf, w3a, w3b, b3r, w4, b4r)


def kernel(node_features, frac_coords, lattices, edge_index, edge2graph,
           W1, b1, W2, b2, W3, b3, W4, b4):
    src = edge_index[0]
    dst = edge_index[1]
    lat9 = lattices.reshape(_G, 9)
    fc16 = jnp.pad(frac_coords, ((0, 0), (0, 13)))
    w1a = W1[:_H]
    w1b = W1[_H:2 * _H]
    w1c = W1[2 * _H:2 * _H + 9]
    w1dp = jnp.pad(W1[2 * _H + 9:], ((0, 13), (0, 0)))
    b1r = b1.reshape(1, _H)
    b2r = b2.reshape(1, _H)
    b3r = b3.reshape(1, _H)
    b4r = b4.reshape(1, _H)

    p, q, rb = _pre(node_features, lat9, w1a, w1b, w1c, b1r)
    a, b, c = _gather_kernel()(p, q, rb, src, dst, edge2graph)
    ones16 = jnp.ones((_W, 16), jnp.float32)
    zc = jnp.zeros((_N, 16), jnp.float32)
    xi, xj, cnt = _aux_kernel()(fc16, src, dst, ones16, zc)
    ef = _edge(a, b, c, xi, xj, w1dp, W2, b2r)
    zeros = jnp.zeros((_N, _H), jnp.float32)
    part = _scatter_kernel()(ef, src, zeros)
    return _node(part, cnt, node_features, W3[:_H], W3[_H:], b3r, W4, b4r)


# speedup vs baseline: 2.5533x; 1.1550x over previous
import functools

import jax
import jax.numpy as jnp
from jax import lax
from jax.experimental import pallas as pl
from jax.experimental.pallas import tpu as pltpu
from jax.experimental.pallas import tpu_sc as plsc

_N = 10000
_E = 320000
_G = 256
_H = 128

_NC = 2
_NS = 16
_NW = _NC * _NS
_EPW = _E // _NW
_W = 80

_BE = 2000
_BN = 1000


@functools.lru_cache(maxsize=None)
def _mesh():
    return plsc.VectorSubcoreMesh(core_axis_name="c", subcore_axis_name="s")


def _pre_body(nf_ref, lat9_ref, w1a_ref, w1b_ref, w1c_ref, b1_ref,
              p_ref, q_ref, rb_ref):
    x = nf_ref[...]
    p_ref[...] = jnp.dot(x, w1a_ref[...], preferred_element_type=jnp.float32)
    q_ref[...] = jnp.dot(x, w1b_ref[...], preferred_element_type=jnp.float32)

    @pl.when(pl.program_id(0) == 0)
    def _():
        lat9 = lat9_ref[...]
        cols = [lat9[:, k:k + 1] for k in range(9)]
        acc = jnp.broadcast_to(b1_ref[...], (_G, _H))
        for i in range(3):
            for j in range(3):
                ip = (cols[3 * i + 0] * cols[3 * j + 0]
                      + cols[3 * i + 1] * cols[3 * j + 1]
                      + cols[3 * i + 2] * cols[3 * j + 2])
                acc = acc + ip * w1c_ref[3 * i + j:3 * i + j + 1, :]
        rb_ref[...] = acc


def _pre(nf, lat9, w1a, w1b, w1c, b1r):
    grid = (_N // _BN,)
    return pl.pallas_call(
        _pre_body,
        grid=grid,
        in_specs=[
            pl.BlockSpec((_BN, _H), lambda i: (i, 0)),
            pl.BlockSpec((_G, 9), lambda i: (0, 0)),
            pl.BlockSpec((_H, _H), lambda i: (0, 0)),
            pl.BlockSpec((_H, _H), lambda i: (0, 0)),
            pl.BlockSpec((9, _H), lambda i: (0, 0)),
            pl.BlockSpec((1, _H), lambda i: (0, 0)),
        ],
        out_specs=[
            pl.BlockSpec((_BN, _H), lambda i: (i, 0)),
            pl.BlockSpec((_BN, _H), lambda i: (i, 0)),
            pl.BlockSpec((_G, _H), lambda i: (0, 0)),
        ],
        out_shape=[
            jax.ShapeDtypeStruct((_N, _H), jnp.float32),
            jax.ShapeDtypeStruct((_N, _H), jnp.float32),
            jax.ShapeDtypeStruct((_G, _H), jnp.float32),
        ],
    )(nf, lat9, w1a, w1b, w1c, b1r)


_NWIN = _EPW // _W


def _gather_body(p_hbm, q_hbm, rb_hbm, src_hbm, dst_hbm, e2g_hbm,
                 a_hbm, b_hbm, c_hbm,
                 si_v, di_v, gi_v, a0_v, b0_v, c0_v, a1_v, b1_v, c1_v,
                 sem0, sem1):
    wid = lax.axis_index("s") * _NC + lax.axis_index("c")
    base = wid * _EPW

    pltpu.sync_copy(src_hbm.at[pl.ds(base, _EPW)], si_v)
    pltpu.sync_copy(dst_hbm.at[pl.ds(base, _EPW)], di_v)
    pltpu.sync_copy(e2g_hbm.at[pl.ds(base, _EPW)], gi_v)

    def issue(w, a_v, b_v, c_v, sem):
        sl = pl.ds(w * _W, _W)
        pltpu.async_copy(p_hbm.at[si_v.at[sl]], a_v, sem)
        pltpu.async_copy(q_hbm.at[di_v.at[sl]], b_v, sem)
        pltpu.async_copy(rb_hbm.at[gi_v.at[sl]], c_v, sem)

    def drain_store(w, a_v, b_v, c_v, sem):
        sl = pl.ds(w * _W, _W)
        pltpu.make_async_copy(p_hbm.at[si_v.at[sl]], a_v, sem).wait()
        pltpu.make_async_copy(q_hbm.at[di_v.at[sl]], b_v, sem).wait()
        pltpu.make_async_copy(rb_hbm.at[gi_v.at[sl]], c_v, sem).wait()
        st = base + w * _W
        pltpu.sync_copy(a_v, a_hbm.at[pl.ds(st, _W)])
        pltpu.sync_copy(b_v, b_hbm.at[pl.ds(st, _W)])
        pltpu.sync_copy(c_v, c_hbm.at[pl.ds(st, _W)])

    issue(0, a0_v, b0_v, c0_v, sem0)

    @pl.loop(0, _NWIN - 1, step=2)
    def _(w):
        issue(w + 1, a1_v, b1_v, c1_v, sem1)
        drain_store(w, a0_v, b0_v, c0_v, sem0)
        issue(w + 2, a0_v, b0_v, c0_v, sem0)
        drain_store(w + 1, a1_v, b1_v, c1_v, sem1)

    drain_store(_NWIN - 1, a0_v, b0_v, c0_v, sem0)


@functools.lru_cache(maxsize=None)
def _gather_kernel():
    return pl.kernel(
        _gather_body,
        mesh=_mesh(),
        out_type=(
            jax.ShapeDtypeStruct((_E, _H), jnp.float32),
            jax.ShapeDtypeStruct((_E, _H), jnp.float32),
            jax.ShapeDtypeStruct((_E, _H), jnp.float32),
        ),
        scratch_types=[
            pltpu.VMEM((_EPW,), jnp.int32),
            pltpu.VMEM((_EPW,), jnp.int32),
            pltpu.VMEM((_EPW,), jnp.int32),
            pltpu.VMEM((_W, _H), jnp.float32),
            pltpu.VMEM((_W, _H), jnp.float32),
            pltpu.VMEM((_W, _H), jnp.float32),
            pltpu.VMEM((_W, _H), jnp.float32),
            pltpu.VMEM((_W, _H), jnp.float32),
            pltpu.VMEM((_W, _H), jnp.float32),
            pltpu.SemaphoreType.DMA,
            pltpu.SemaphoreType.DMA,
        ],
    )


def _aux_body(fc_hbm, src_hbm, dst_hbm, ones_hbm, zc_hbm,
              xi_hbm, xj_hbm, cnt_hbm,
              si_v, di_v, xi_v, xj_v, ones_v, cacc_sh, sem):
    cid = lax.axis_index("c")
    sid = lax.axis_index("s")
    wid = sid * _NC + cid
    base = wid * _EPW

    @pl.when(sid == 0)
    def _():
        pltpu.sync_copy(zc_hbm, cacc_sh)
    pltpu.sync_copy(ones_hbm, ones_v)
    plsc.subcore_barrier()

    @pl.loop(0, _EPW, step=_W)
    def _(off):
        st = base + off
        pltpu.sync_copy(src_hbm.at[pl.ds(st, _W)], si_v)
        pltpu.sync_copy(dst_hbm.at[pl.ds(st, _W)], di_v)
        c1 = pltpu.async_copy(fc_hbm.at[si_v], xi_v, sem)
        c2 = pltpu.async_copy(fc_hbm.at[di_v], xj_v, sem)
        c1.wait()
        c2.wait()
        pltpu.sync_copy(xi_v, xi_hbm.at[pl.ds(st, _W)])
        pltpu.sync_copy(xj_v, xj_hbm.at[pl.ds(st, _W)])
        pltpu.sync_copy(ones_v, cacc_sh.at[si_v], add=True)

    plsc.subcore_barrier()

    @pl.when(sid == 0)
    def _():
        pltpu.sync_copy(cacc_sh, cnt_hbm.at[cid])


@functools.lru_cache(maxsize=None)
def _aux_kernel():
    return pl.kernel(
        _aux_body,
        mesh=_mesh(),
        out_type=(
            jax.ShapeDtypeStruct((_E, 16), jnp.float32),
            jax.ShapeDtypeStruct((_E, 16), jnp.float32),
            jax.ShapeDtypeStruct((_NC, _N, 16), jnp.float32),
        ),
        scratch_types=[
            pltpu.VMEM((_W,), jnp.int32),
            pltpu.VMEM((_W,), jnp.int32),
            pltpu.VMEM((_W, 16), jnp.float32),
            pltpu.VMEM((_W, 16), jnp.float32),
            pltpu.VMEM((_W, 16), jnp.float32),
            pltpu.VMEM_SHARED((_N, 16), jnp.float32),
            pltpu.SemaphoreType.DMA,
        ],
        compiler_params=pltpu.CompilerParams(use_tc_tiling_on_sc=False),
    )


def _edge_body(a_ref, b_ref, c_ref, xi_ref, xj_ref, w1dp_ref, w2_ref, b2_ref,
               out_ref):
    z = a_ref[...] + b_ref[...] + c_ref[...]
    d = xj_ref[...] - xi_ref[...]
    fd = jnp.where(d < 0.0, d + 1.0, d)
    z = z + jnp.dot(fd, w1dp_ref[...], preferred_element_type=jnp.float32)
    u = z * jax.nn.sigmoid(z)
    h2 = jnp.dot(u, w2_ref[...], preferred_element_type=jnp.float32) + b2_ref[...]
    out_ref[...] = h2 * jax.nn.sigmoid(h2)


def _edge(a, b, c, xi, xj, w1dp, w2, b2r):
    grid = (_E // _BE,)
    return pl.pallas_call(
        _edge_body,
        grid=grid,
        in_specs=[
            pl.BlockSpec((_BE, _H), lambda i: (i, 0)),
            pl.BlockSpec((_BE, _H), lambda i: (i, 0)),
            pl.BlockSpec((_BE, _H), lambda i: (i, 0)),
            pl.BlockSpec((_BE, 16), lambda i: (i, 0)),
            pl.BlockSpec((_BE, 16), lambda i: (i, 0)),
            pl.BlockSpec((16, _H), lambda i: (0, 0)),
            pl.BlockSpec((_H, _H), lambda i: (0, 0)),
            pl.BlockSpec((1, _H), lambda i: (0, 0)),
        ],
        out_specs=pl.BlockSpec((_BE, _H), lambda i: (i, 0)),
        out_shape=jax.ShapeDtypeStruct((_E, _H), jnp.float32),
    )(a, b, c, xi, xj, w1dp, w2, b2r)


def _scatter_body(ef_hbm, src_hbm, z_hbm, part_hbm, idx_v, rows_v, acc_sh,
                  sem):
    cid = lax.axis_index("c")
    sid = lax.axis_index("s")
    wid = sid * _NC + cid
    base = wid * _EPW

    @pl.when(sid == 0)
    def _():
        pltpu.sync_copy(z_hbm, acc_sh)
    plsc.subcore_barrier()

    @pl.loop(0, _EPW, step=_W)
    def _(off):
        st = base + off
        pltpu.sync_copy(src_hbm.at[pl.ds(st, _W)], idx_v)
        pltpu.sync_copy(ef_hbm.at[pl.ds(st, _W)], rows_v)
        pltpu.sync_copy(rows_v, acc_sh.at[idx_v], add=True)

    plsc.subcore_barrier()

    @pl.when(sid == 0)
    def _():
        pltpu.sync_copy(acc_sh, part_hbm.at[cid])


@functools.lru_cache(maxsize=None)
def _scatter_kernel():
    return pl.kernel(
        _scatter_body,
        mesh=_mesh(),
        out_type=jax.ShapeDtypeStruct((_NC, _N, _H), jnp.float32),
        scratch_types=[
            pltpu.VMEM((_W,), jnp.int32),
            pltpu.VMEM((_W, _H), jnp.float32),
            pltpu.VMEM_SHARED((_N, _H), jnp.float32),
            pltpu.SemaphoreType.DMA,
        ],
    )


def _node_body(part_ref, cnt_ref, nf_ref, w3a_ref, w3b_ref, b3_ref, w4_ref,
               b4_ref, out_ref):
    s = part_ref[0] + part_ref[1]
    cnt = cnt_ref[0][:, 0:1] + cnt_ref[1][:, 0:1]
    agg = s / jnp.maximum(cnt, 1.0)
    x = nf_ref[...]
    h = (jnp.dot(x, w3a_ref[...], preferred_element_type=jnp.float32)
         + jnp.dot(agg, w3b_ref[...], preferred_element_type=jnp.float32)
         + b3_ref[...])
    u = h * jax.nn.sigmoid(h)
    h2 = jnp.dot(u, w4_ref[...], preferred_element_type=jnp.float32) + b4_ref[...]
    out_ref[...] = x + h2 * jax.nn.sigmoid(h2)


def _node(part, cnt, nf, w3a, w3b, b3r, w4, b4r):
    grid = (_N // _BN,)
    return pl.pallas_call(
        _node_body,
        grid=grid,
        in_specs=[
            pl.BlockSpec((_NC, _BN, _H), lambda i: (0, i, 0)),
            pl.BlockSpec((_NC, _BN, 16), lambda i: (0, i, 0)),
            pl.BlockSpec((_BN, _H), lambda i: (i, 0)),
            pl.BlockSpec((_H, _H), lambda i: (0, 0)),
            pl.BlockSpec((_H, _H), lambda i: (0, 0)),
            pl.BlockSpec((1, _H), lambda i: (0, 0)),
            pl.BlockSpec((_H, _H), lambda i: (0, 0)),
            pl.BlockSpec((1, _H), lambda i: (0, 0)),
        ],
        out_specs=pl.BlockSpec((_BN, _H), lambda i: (i, 0)),
        out_shape=jax.ShapeDtypeStruct((_N, _H), jnp.float32),
    )(part, cnt, nf, w3a, w3b, b3r, w4, b4r)


def kernel(node_features, frac_coords, lattices, edge_index, edge2graph,
           W1, b1, W2, b2, W3, b3, W4, b4):
    src = edge_index[0]
    dst = edge_index[1]
    lat9 = lattices.reshape(_G, 9)
    fc16 = jnp.pad(frac_coords, ((0, 0), (0, 13)))
    w1a = W1[:_H]
    w1b = W1[_H:2 * _H]
    w1c = W1[2 * _H:2 * _H + 9]
    w1dp = jnp.pad(W1[2 * _H + 9:], ((0, 13), (0, 0)))
    b1r = b1.reshape(1, _H)
    b2r = b2.reshape(1, _H)
    b3r = b3.reshape(1, _H)
    b4r = b4.reshape(1, _H)

    p, q, rb = _pre(node_features, lat9, w1a, w1b, w1c, b1r)
    a, b, c = _gather_kernel()(p, q, rb, src, dst, edge2graph)
    ones16 = jnp.ones((_W, 16), jnp.float32)
    zc = jnp.zeros((_N, 16), jnp.float32)
    xi, xj, cnt = _aux_kernel()(fc16, src, dst, ones16, zc)
    ef = _edge(a, b, c, xi, xj, w1dp, W2, b2r)
    zeros = jnp.zeros((_N, _H), jnp.float32)
    part = _scatter_kernel()(ef, src, zeros)
    return _node(part, cnt, node_features, W3[:_H], W3[_H:], b3r, W4, b4r)
